# baseline (speedup 1.0000x reference)
import functools

import numpy as np
import jax
import jax.numpy as jnp
from jax import lax
from jax.experimental import pallas as pl
from jax.experimental.pallas import tpu as pltpu
from jax.experimental.pallas import tpu_sc as plsc

_SAMPLE = 20
_CHUNK = 256
_HALF = _CHUNK // 2


def _threefry2x32(k0, k1, x0, x1):
    x0 = np.asarray(x0, np.uint32).copy()
    x1 = np.asarray(x1, np.uint32).copy()
    k0 = np.uint32(k0)
    k1 = np.uint32(k1)
    ks = [k0, k1, np.uint32(k0 ^ k1 ^ np.uint32(0x1BD11BDA))]
    rot = [(13, 15, 26, 6), (17, 29, 16, 24)]
    x0 = (x0 + ks[0]).astype(np.uint32)
    x1 = (x1 + ks[1]).astype(np.uint32)
    for i in range(5):
        for r in rot[i % 2]:
            x0 = (x0 + x1).astype(np.uint32)
            x1 = ((x1 << np.uint32(r)) | (x1 >> np.uint32(32 - r))).astype(np.uint32)
            x1 = x0 ^ x1
        x0 = (x0 + ks[(i + 1) % 3]).astype(np.uint32)
        x1 = (x1 + ks[(i + 2) % 3] + np.uint32(i + 1)).astype(np.uint32)
    return x0, x1


def _np_randint_key42(n: int, maxval: int) -> np.ndarray:
    s1, s2 = _threefry2x32(0, 42, np.zeros(2, np.uint32), np.arange(2, dtype=np.uint32))
    zero = np.zeros(n, np.uint32)
    iota = np.arange(n, dtype=np.uint32)
    h1, h2 = _threefry2x32(s1[0], s2[0], zero, iota)
    l1, l2 = _threefry2x32(s1[1], s2[1], zero, iota)
    hi, lo = h1 ^ h2, l1 ^ l2
    span = np.uint32(maxval)
    m = np.uint32(np.uint32(65536) % span)
    mult = np.uint32(np.uint32(m * m) % span)
    off = ((hi % span).astype(np.uint32) * mult + (lo % span)) % span
    return off.astype(np.int32)


@functools.lru_cache(maxsize=None)
def _work_indices(bs: int, nw: int):
    ridx = _np_randint_key42(_SAMPLE * bs, bs).astype(np.int64)

    p = np.arange(bs, dtype=np.int64).reshape(nw, -1, _HALF)
    pos_items = np.stack([p, bs + p], axis=2)

    q = np.arange(_SAMPLE * bs, dtype=np.int64).reshape(nw, -1, _HALF)
    neg_items = np.stack([q // _SAMPLE, bs + ridx[q]], axis=2)

    n_items = pos_items.shape[1] + neg_items.shape[1]
    n_pad = -n_items % 8
    widx = np.concatenate(
        [
            pos_items.reshape(nw, -1, _CHUNK),
            neg_items.reshape(nw, -1, _CHUNK),
            np.zeros((nw, n_pad, _CHUNK), np.int64),
        ],
        axis=1,
    ).astype(np.int32)
    return jnp.asarray(widx.reshape(-1))


@functools.lru_cache(maxsize=None)
def _build_kernel(bs: int):
    info = plsc.get_sparse_core_info()
    nw = info.num_cores * info.num_subcores

    pos_per_w = bs // nw
    neg_per_w = _SAMPLE * bs // nw
    pos_items = pos_per_w // _HALF
    neg_items = neg_per_w // _HALF
    n_items = pos_items + neg_items
    n_padded = n_items + (-n_items % 8)

    mesh = plsc.VectorSubcoreMesh(core_axis_name="c", subcore_axis_name="s")

    @functools.partial(
        pl.kernel,
        mesh=mesh,
        out_type=(
            jax.ShapeDtypeStruct((2, 8, bs // 128, 8, 128), jnp.float32),
            jax.ShapeDtypeStruct((2, 8, _SAMPLE * bs // 128, 8, 128), jnp.float32),
        ),
        scratch_types=[
            pltpu.VMEM((n_padded * _CHUNK,), jnp.int32),
            pltpu.VMEM((_CHUNK, 64), jnp.float32),
            pltpu.VMEM((_CHUNK, 64), jnp.float32),
            pltpu.VMEM((8, 8, _CHUNK), jnp.float32),
            pltpu.VMEM((8, 8, _CHUNK), jnp.float32),
            pltpu.SemaphoreType.DMA,
            pltpu.SemaphoreType.DMA,
            pltpu.SemaphoreType.DMA,
            pltpu.SemaphoreType.DMA,
        ],
        compiler_params=pltpu.CompilerParams(
            needs_layout_passes=False,
            disable_bounds_checks=True,
            use_tc_tiling_on_sc=False,
        ),
    )
    def k(table_hbm, widx_hbm, pos_out, neg_out,
          idx_v, g0, g1, t0, t1, gs0, gs1, ws0, ws1):
        wid = lax.axis_index("s") * info.num_cores + lax.axis_index("c")
        gbufs = (g0, g1)
        tbufs = (t0, t1)
        gsems = (gs0, gs1)
        wsems = (ws0, ws1)

        nw_idx = n_padded * _CHUNK
        pltpu.sync_copy(widx_hbm.at[pl.ds(wid * nw_idx, nw_idx)], idx_v)

        def start_gather(item, b):
            pltpu.async_copy(
                table_hbm.at[idx_v.at[pl.ds(item * _CHUNK, _CHUNK)]],
                gbufs[b], gsems[b],
            )

        def wait_gather(b):
            pltpu.make_async_copy(table_hbm.at[pl.ds(0, _CHUNK)], gbufs[b], gsems[b]).wait()

        kiota = lax.iota(jnp.int32, 16)
        riota = [kiota + 16 * u for u in range(_CHUNK // 16)]

        def transpose_chunk(b):
            g, t = gbufs[b], tbufs[b]

            @plsc.parallel_loop(0, 64, unroll=8)
            def _t(c):
                cvec = (kiota + c) & 63
                ahi = cvec >> 3
                alo = cvec & 7
                for u in range(_CHUNK // 16):
                    v = plsc.load_gather(g, [riota[u], cvec])
                    plsc.store_scatter(t, [ahi, alo, riota[u]], v)

        def write_chunk(out, kblk, b):
            pltpu.async_copy(tbufs[b].at[:, :, pl.ds(0, _HALF)],
                             out.at[0, :, kblk], wsems[b])
            pltpu.async_copy(tbufs[b].at[:, :, pl.ds(_HALF, _HALF)],
                             out.at[1, :, kblk], wsems[b])

        def wait_write(out, kblk, b):
            pltpu.make_async_copy(tbufs[b].at[:, :, pl.ds(0, _HALF)],
                                  out.at[0, :, kblk], wsems[b]).wait()
            pltpu.make_async_copy(tbufs[b].at[:, :, pl.ds(_HALF, _HALF)],
                                  out.at[1, :, kblk], wsems[b]).wait()

        for item in range(pos_items):
            b = item % 2
            start_gather(item, b)
            wait_gather(b)
            kblk = wid * pos_items + item
            transpose_chunk(b)
            write_chunk(pos_out, kblk, b)
            wait_write(pos_out, kblk, b)

        nkbase = wid * neg_items

        def gidx(i):
            return pos_items + i

        start_gather(gidx(0), 0)
        start_gather(gidx(1), 1)

        def body(g, carry):
            for b in range(2):
                item = 2 * g + b
                wait_gather(b)

                @pl.when(item >= 2)
                def _drain(item=item, b=b):
                    wait_write(neg_out, nkbase + item - 2, b)

                transpose_chunk(b)
                write_chunk(neg_out, nkbase + item, b)

                @pl.when(item + 2 < neg_items)
                def _prefetch(item=item, b=b):
                    start_gather(gidx(item + 2), b)
            return carry

        lax.fori_loop(0, neg_items // 2, body, 0)
        wait_write(neg_out, nkbase + neg_items - 2, 0)
        wait_write(neg_out, nkbase + neg_items - 1, 1)

    return k


def kernel(anchor, target):
    bs, d = target.shape
    info = plsc.get_sparse_core_info()
    nw = info.num_cores * info.num_subcores
    table = jnp.concatenate([anchor, target], axis=0)
    widx = _work_indices(bs, nw)
    pos5, neg5 = _build_kernel(bs)(table, widx)
    pos = jnp.transpose(pos5, (2, 4, 0, 1, 3)).reshape(bs, 2, d)
    neg = jnp.transpose(neg5, (2, 4, 0, 1, 3)).reshape(_SAMPLE * bs, 2, d)
    return pos, neg


# speedup vs baseline: 1.0175x; 1.0175x over previous
import functools

import numpy as np
import jax
import jax.numpy as jnp
from jax import lax
from jax.experimental import pallas as pl
from jax.experimental.pallas import tpu as pltpu
from jax.experimental.pallas import tpu_sc as plsc

_SAMPLE = 20
_CHUNK = 256
_HALF = _CHUNK // 2


def _threefry2x32(k0, k1, x0, x1):
    x0 = np.asarray(x0, np.uint32).copy()
    x1 = np.asarray(x1, np.uint32).copy()
    k0 = np.uint32(k0)
    k1 = np.uint32(k1)
    ks = [k0, k1, np.uint32(k0 ^ k1 ^ np.uint32(0x1BD11BDA))]
    rot = [(13, 15, 26, 6), (17, 29, 16, 24)]
    x0 = (x0 + ks[0]).astype(np.uint32)
    x1 = (x1 + ks[1]).astype(np.uint32)
    for i in range(5):
        for r in rot[i % 2]:
            x0 = (x0 + x1).astype(np.uint32)
            x1 = ((x1 << np.uint32(r)) | (x1 >> np.uint32(32 - r))).astype(np.uint32)
            x1 = x0 ^ x1
        x0 = (x0 + ks[(i + 1) % 3]).astype(np.uint32)
        x1 = (x1 + ks[(i + 2) % 3] + np.uint32(i + 1)).astype(np.uint32)
    return x0, x1


def _np_randint_key42(n: int, maxval: int) -> np.ndarray:
    s1, s2 = _threefry2x32(0, 42, np.zeros(2, np.uint32), np.arange(2, dtype=np.uint32))
    zero = np.zeros(n, np.uint32)
    iota = np.arange(n, dtype=np.uint32)
    h1, h2 = _threefry2x32(s1[0], s2[0], zero, iota)
    l1, l2 = _threefry2x32(s1[1], s2[1], zero, iota)
    hi, lo = h1 ^ h2, l1 ^ l2
    span = np.uint32(maxval)
    m = np.uint32(np.uint32(65536) % span)
    mult = np.uint32(np.uint32(m * m) % span)
    off = ((hi % span).astype(np.uint32) * mult + (lo % span)) % span
    return off.astype(np.int32)


@functools.lru_cache(maxsize=None)
def _work_indices(bs: int, nw: int):
    ridx = _np_randint_key42(_SAMPLE * bs, bs).astype(np.int64)

    p = np.arange(bs, dtype=np.int64).reshape(nw, -1, _HALF)
    pos_items = np.stack([p, bs + p], axis=2)

    q = np.arange(_SAMPLE * bs, dtype=np.int64).reshape(nw, -1, _HALF)
    neg_items = np.stack([q // _SAMPLE, bs + ridx[q]], axis=2)

    n_items = pos_items.shape[1] + neg_items.shape[1]
    n_pad = -n_items % 8
    widx = np.concatenate(
        [
            pos_items.reshape(nw, -1, _CHUNK),
            neg_items.reshape(nw, -1, _CHUNK),
            np.zeros((nw, n_pad, _CHUNK), np.int64),
        ],
        axis=1,
    ).astype(np.int32)
    return jnp.asarray(widx.reshape(-1))


@functools.lru_cache(maxsize=None)
def _build_kernel(bs: int):
    info = plsc.get_sparse_core_info()
    nw = info.num_cores * info.num_subcores

    pos_per_w = bs // nw
    neg_per_w = _SAMPLE * bs // nw
    pos_items = pos_per_w // _HALF
    neg_items = neg_per_w // _HALF
    n_items = pos_items + neg_items
    n_padded = n_items + (-n_items % 8)

    mesh = plsc.VectorSubcoreMesh(core_axis_name="c", subcore_axis_name="s")

    @functools.partial(
        pl.kernel,
        mesh=mesh,
        out_type=(
            jax.ShapeDtypeStruct((2, 8, bs // 128, 8, 128), jnp.float32),
            jax.ShapeDtypeStruct((2, 8, _SAMPLE * bs // 128, 8, 128), jnp.float32),
        ),
        scratch_types=[
            pltpu.VMEM((n_padded * _CHUNK,), jnp.int32),
            pltpu.VMEM((_CHUNK, 64), jnp.float32),
            pltpu.VMEM((_CHUNK, 64), jnp.float32),
            pltpu.VMEM((8, 8, _CHUNK), jnp.float32),
            pltpu.VMEM((8, 8, _CHUNK), jnp.float32),
            pltpu.SemaphoreType.DMA,
            pltpu.SemaphoreType.DMA,
            pltpu.SemaphoreType.DMA,
            pltpu.SemaphoreType.DMA,
        ],
        compiler_params=pltpu.CompilerParams(
            needs_layout_passes=False,
            disable_bounds_checks=True,
            use_tc_tiling_on_sc=False,
        ),
    )
    def k(table_hbm, widx_hbm, pos_out, neg_out,
          idx_v, g0, g1, t0, t1, gs0, gs1, ws0, ws1):
        wid = lax.axis_index("s") * info.num_cores + lax.axis_index("c")
        gbufs = (g0, g1)
        tbufs = (t0, t1)
        gsems = (gs0, gs1)
        wsems = (ws0, ws1)

        nw_idx = n_padded * _CHUNK
        pltpu.sync_copy(widx_hbm.at[pl.ds(wid * nw_idx, nw_idx)], idx_v)

        def start_gather(item, b):
            pltpu.async_copy(
                table_hbm.at[idx_v.at[pl.ds(item * _CHUNK, _CHUNK)]],
                gbufs[b], gsems[b],
            )

        def wait_gather(b):
            pltpu.make_async_copy(table_hbm.at[pl.ds(0, _CHUNK)], gbufs[b], gsems[b]).wait()

        kiota = lax.iota(jnp.int32, 16)
        riota = [kiota + 16 * u for u in range(_CHUNK // 16)]

        def transpose_chunk(b):
            g, t = gbufs[b], tbufs[b]

            @plsc.parallel_loop(0, 64, unroll=4)
            def _t(c):
                cvec = (kiota + c) & 63
                ahi = cvec >> 3
                alo = cvec & 7
                for u in range(_CHUNK // 16):
                    v = plsc.load_gather(g, [riota[u], cvec])
                    plsc.store_scatter(t, [ahi, alo, riota[u]], v)

        def write_chunk(out, kblk, b):
            pltpu.async_copy(tbufs[b].at[:, :, pl.ds(0, _HALF)],
                             out.at[0, :, kblk], wsems[b])
            pltpu.async_copy(tbufs[b].at[:, :, pl.ds(_HALF, _HALF)],
                             out.at[1, :, kblk], wsems[b])

        def wait_write(out, kblk, b):
            pltpu.make_async_copy(tbufs[b].at[:, :, pl.ds(0, _HALF)],
                                  out.at[0, :, kblk], wsems[b]).wait()
            pltpu.make_async_copy(tbufs[b].at[:, :, pl.ds(_HALF, _HALF)],
                                  out.at[1, :, kblk], wsems[b]).wait()

        for item in range(pos_items):
            b = item % 2
            start_gather(item, b)
            wait_gather(b)
            kblk = wid * pos_items + item
            transpose_chunk(b)
            write_chunk(pos_out, kblk, b)
            wait_write(pos_out, kblk, b)

        nkbase = wid * neg_items

        def gidx(i):
            return pos_items + i

        start_gather(gidx(0), 0)
        start_gather(gidx(1), 1)

        def body(g, carry):
            for b in range(2):
                item = 2 * g + b
                wait_gather(b)

                @pl.when(item >= 2)
                def _drain(item=item, b=b):
                    wait_write(neg_out, nkbase + item - 2, b)

                transpose_chunk(b)
                write_chunk(neg_out, nkbase + item, b)

                @pl.when(item + 2 < neg_items)
                def _prefetch(item=item, b=b):
                    start_gather(gidx(item + 2), b)
            return carry

        lax.fori_loop(0, neg_items // 2, body, 0)
        wait_write(neg_out, nkbase + neg_items - 2, 0)
        wait_write(neg_out, nkbase + neg_items - 1, 1)

    return k


def kernel(anchor, target):
    bs, d = target.shape
    info = plsc.get_sparse_core_info()
    nw = info.num_cores * info.num_subcores
    table = jnp.concatenate([anchor, target], axis=0)
    widx = _work_indices(bs, nw)
    pos5, neg5 = _build_kernel(bs)(table, widx)
    pos = jnp.transpose(pos5, (2, 4, 0, 1, 3)).reshape(bs, 2, d)
    neg = jnp.transpose(neg5, (2, 4, 0, 1, 3)).reshape(_SAMPLE * bs, 2, d)
    return pos, neg
